# baseline (speedup 1.0000x reference)
import functools

import numpy as np
import jax
import jax.numpy as jnp
from jax import lax
from jax.experimental import pallas as pl
from jax.experimental.pallas import tpu as pltpu
from jax.experimental.pallas import tpu_sc as plsc

_VOCAB = 1000000
_EMBED = 64
_BATCH = 4096
_SEQLEN = 200

_NC = 2
_NS = 16
_NW = _NC * _NS
_TOTAL = _BATCH * _SEQLEN
_PER_W = _TOTAL // _NW
_C = 800
_NCH = _PER_W // _C
_LANE = 16
_KG = _EMBED // _LANE


def _pe_rows():
    pos = np.arange(_SEQLEN, dtype=np.float32)[:, None]
    div = np.exp(
        np.arange(0, _EMBED, 2, dtype=np.float32) * (-np.log(10000.0) / _EMBED)
    )
    pe = np.zeros((_SEQLEN, _EMBED), dtype=np.float32)
    pe[:, 0::2] = np.sin(pos * div)
    pe[:, 1::2] = np.cos(pos * div)
    return jnp.asarray(pe)


def _make_kernel():
    mesh = plsc.VectorSubcoreMesh(core_axis_name="c", subcore_axis_name="s")

    @functools.partial(
        pl.kernel,
        mesh=mesh,
        out_type=jax.ShapeDtypeStruct((_TOTAL, _EMBED), jnp.float32),
        compiler_params=pltpu.CompilerParams(use_tc_tiling_on_sc=False),
        scratch_types=[
            pltpu.VMEM((_SEQLEN, _EMBED), jnp.float32),
            pltpu.VMEM((_C,), jnp.int32),
            pltpu.VMEM((_C,), jnp.int32),
            pltpu.VMEM((_C, _EMBED), jnp.float32),
            pltpu.VMEM((_C, _EMBED), jnp.float32),
            pltpu.SemaphoreType.DMA,
            pltpu.SemaphoreType.DMA,
            pltpu.SemaphoreType.DMA,
            pltpu.SemaphoreType.DMA,
            pltpu.SemaphoreType.DMA,
            pltpu.SemaphoreType.DMA,
        ],
    )
    def emb_kernel(xf, pe_h, tbl, out, pe_v, idx0, idx1, rows0, rows1,
                   gsem0, gsem1, osem0, osem1, isem0, isem1):
        wid = lax.axis_index("s") * _NC + lax.axis_index("c")
        base = wid * _PER_W

        pltpu.sync_copy(pe_h, pe_v)

        idx = [idx0, idx1]
        rows = [rows0, rows1]
        gsem = [gsem0, gsem1]
        osem = [osem0, osem1]
        isem = [isem0, isem1]

        def add_pe(rref):
            def l_body(l, carry):
                for k in range(_KG):
                    pek = pe_v[l, pl.ds(k * _LANE, _LANE)]
                    for rep in range(_C // _SEQLEN):
                        r = rep * _SEQLEN + l
                        sl = pl.ds(k * _LANE, _LANE)
                        rref[r, sl] = rref[r, sl] + pek
                return carry
            lax.fori_loop(0, _SEQLEN, l_body, 0)

        pltpu.sync_copy(xf.at[pl.ds(base, _C)], idx[0])
        g_h = [None, None]
        o_h = [None, None]
        g_h[0] = pltpu.async_copy(tbl.at[idx[0]], rows[0], gsem[0])
        i_h = pltpu.async_copy(xf.at[pl.ds(base + _C, _C)], idx[1], isem[1])

        for g in range(_NCH):
            cur = g & 1
            nxt = cur ^ 1
            g_h[cur].wait()
            if g + 1 < _NCH:
                i_h.wait()
                if o_h[nxt] is not None:
                    o_h[nxt].wait()
                g_h[nxt] = pltpu.async_copy(tbl.at[idx[nxt]], rows[nxt],
                                            gsem[nxt])
                if g + 2 < _NCH:
                    i_h = pltpu.async_copy(
                        xf.at[pl.ds(base + (g + 2) * _C, _C)], idx[cur],
                        isem[cur])
            add_pe(rows[cur])
            o_h[cur] = pltpu.async_copy(rows[cur],
                                        out.at[pl.ds(base + g * _C, _C)],
                                        osem[cur])
        o_h[0].wait()
        o_h[1].wait()

    return emb_kernel


_EMB_KERNEL = _make_kernel()


def kernel(x, table):
    xf = x.reshape(_TOTAL)
    out = _EMB_KERNEL(xf, _pe_rows(), table)
    return out.reshape(_BATCH, _SEQLEN, _EMBED)
